# baseline (speedup 1.0000x reference)
import functools

import jax
import jax.numpy as jnp
from jax import lax
from jax.experimental import pallas as pl
from jax.experimental.pallas import tpu as pltpu
from jax.experimental.pallas import tpu_sc as plsc

_NC, _NS, _L = 2, 16, 16
_NW = _NC * _NS


@functools.lru_cache(maxsize=None)
def _build(B, F, V, D):
    fpg = 2
    rows_c = fpg * 128
    qpc = fpg * D
    tq_c = qpc // 8
    tb_w = (B // 128) // _NW
    n_g = F // fpg
    w_rows = tb_w * F
    assert F % fpg == 0 and (B // 128) % _NW == 0 and w_rows % 8 == 0
    assert D % _L == 0 and qpc % 8 == 0

    mesh = plsc.VectorSubcoreMesh(core_axis_name="c", subcore_axis_name="s")

    @functools.partial(
        pl.kernel,
        mesh=mesh,
        compiler_params=pltpu.CompilerParams(
            use_tc_tiling_on_sc=False, needs_layout_passes=False),
        out_type=jax.ShapeDtypeStruct(((F * D) // 8, B // 128, 8, 128),
                                      jnp.float32),
        scratch_types=[
            pltpu.VMEM((w_rows, 128), jnp.int32),
            pltpu.VMEM((F, 128), jnp.int32),
            pltpu.VMEM((2, rows_c, D), jnp.float32),
            pltpu.VMEM((2, tq_c, 8, 128), jnp.float32),
            pltpu.VMEM_SHARED((F * V, D), jnp.float32),
            pltpu.SemaphoreType.DMA,
            pltpu.SemaphoreType.DMA,
        ],
    )
    def tok(idx_hbm, off_hbm, tab_hbm, out_hbm, idx_v, off_v, ga_v, tb_v,
            tab_s, gsem, osem):
        wid = lax.axis_index("s") * _NC + lax.axis_index("c")
        @pl.when(lax.axis_index("s") == 0)
        def _():
            pltpu.sync_copy(tab_hbm, tab_s)
        pltpu.sync_copy(off_hbm, off_v)
        pltpu.sync_copy(idx_hbm.at[pl.ds(wid * w_rows, w_rows)], idx_v)
        plsc.subcore_barrier()
        tb0 = wid * tb_w
        iota = lax.iota(jnp.int32, _L)

        def add_body(r, _):
            f = lax.rem(r, F)
            for k in range(128 // _L):
                s = pl.ds(k * _L, _L)
                idx_v[r, s] = idx_v[r, s] + off_v[f, s]
            return 0

        lax.fori_loop(0, w_rows, add_body, 0)

        def repack(buf):
            def fc_body(fc, _):
                f_l = fc // (128 // _L)
                cb0 = lax.rem(fc, 128 // _L)
                rowv = f_l * 128 + cb0 * _L + iota

                def qf_body(qf, colv):
                    vals = plsc.load_gather(ga_v.at[buf], [rowv, colv])
                    q_local = f_l * D + qf
                    tb_v[buf, q_local // 8, lax.rem(q_local, 8),
                         pl.ds(cb0 * _L, _L)] = vals
                    return colv + 1

                lax.fori_loop(0, D, qf_body, jnp.zeros((_L,), jnp.int32))
                return 0

            lax.fori_loop(0, fpg * (128 // _L), fc_body, 0)

        gd = [None, None]
        wd = [None, None]
        ci = 0
        for tb_l in range(tb_w):
            for g in range(n_g):
                buf = ci % 2
                gd[buf] = [
                    pltpu.async_copy(
                        tab_s.at[idx_v.at[tb_l * F + g * fpg + f_l]],
                        ga_v.at[buf, pl.ds(f_l * 128, 128)],
                        gsem,
                    )
                    for f_l in range(fpg)
                ]
                if ci >= 1:
                    pci = ci - 1
                    pbuf = pci % 2
                    p_tb, p_g = divmod(pci, n_g)
                    if wd[pbuf] is not None:
                        wd[pbuf].wait()
                    for cp in gd[pbuf]:
                        cp.wait()
                    repack(pbuf)
                    wd[pbuf] = pltpu.async_copy(
                        tb_v.at[pbuf],
                        out_hbm.at[pl.ds((p_g * fpg * D) // 8, tq_c),
                                   tb0 + p_tb],
                        osem)
                ci += 1
        pci = ci - 1
        pbuf = pci % 2
        p_tb, p_g = divmod(pci, n_g)
        if wd[pbuf] is not None:
            wd[pbuf].wait()
        for cp in gd[pbuf]:
            cp.wait()
        repack(pbuf)
        wd[pbuf] = pltpu.async_copy(
            tb_v.at[pbuf],
            out_hbm.at[pl.ds((p_g * fpg * D) // 8, tq_c), tb0 + p_tb],
            osem)
        for b in range(2):
            if wd[b] is not None:
                wd[b].wait()

    return tok


def kernel(indices, tables):
    B, F = indices.shape
    F2, V, D = tables.shape
    assert F2 == F
    tok = _build(B, F, V, D)
    idx2 = (indices.astype(jnp.int32).reshape(B // 128, 128, F)
            .transpose(0, 2, 1).reshape((B * F) // 128, 128))
    off = jnp.broadcast_to((jnp.arange(F, dtype=jnp.int32) * V)[:, None],
                           (F, 128))
    out4 = tok(idx2, off, tables.reshape(F * V, D))
    return (out4.transpose(1, 3, 0, 2).reshape(B, F * D))


# speedup vs baseline: 1.3504x; 1.3504x over previous
import functools

import jax
import jax.numpy as jnp
from jax import lax
from jax.experimental import pallas as pl
from jax.experimental.pallas import tpu as pltpu
from jax.experimental.pallas import tpu_sc as plsc

_NC, _NS, _L = 2, 16, 16
_NW = _NC * _NS


@functools.lru_cache(maxsize=None)
def _build(B, F, V, D):
    fq = 128 // D
    Fp = -(-F // fq) * fq
    ntq = Fp // fq
    blk_rows = 8 * Fp
    rows_w = (B // 8 // _NW) * blk_rows
    chunk = 4 * blk_rows
    chunks = rows_w // chunk
    assert rows_w % chunk == 0 and chunk % _L == 0 and (B // 8) % _NW == 0

    mesh = plsc.VectorSubcoreMesh(core_axis_name="c", subcore_axis_name="s")

    @functools.partial(
        pl.kernel,
        mesh=mesh,
        compiler_params=pltpu.CompilerParams(use_tc_tiling_on_sc=False),
        out_type=jax.ShapeDtypeStruct((B * Fp, D), jnp.float32),
        scratch_types=[
            pltpu.VMEM((2, chunk), jnp.int32),
            pltpu.VMEM((chunk,), jnp.int32),
            pltpu.VMEM((2, chunk, D), jnp.float32),
            pltpu.VMEM_SHARED((F * V, D), jnp.float32),
            pltpu.SemaphoreType.DMA,
            pltpu.SemaphoreType.DMA,
        ],
    )
    def tok(idx_hbm, off_hbm, tab_hbm, out_hbm, idx_v, off_v, ga_v, tab_s,
            gsem, osem):
        wid = lax.axis_index("s") * _NC + lax.axis_index("c")
        @pl.when(lax.axis_index("s") == 0)
        def _():
            pltpu.sync_copy(tab_hbm, tab_s)
        pltpu.sync_copy(off_hbm, off_v)
        plsc.subcore_barrier()
        base = wid * rows_w

        def fire_out(c):
            return pltpu.async_copy(
                ga_v.at[c % 2],
                out_hbm.at[pl.ds(base + c * chunk, chunk)],
                osem)

        gd = [None, None]
        od = [None, None]
        for c in range(chunks):
            b = c % 2
            if od[b] is not None:
                od[b].wait()
                od[b] = None
            pltpu.sync_copy(
                idx_hbm.at[pl.ds(base + c * chunk, chunk)], idx_v.at[b])

            def add_body(k, _):
                s = pl.ds(k * _L, _L)
                idx_v[b, s] = idx_v[b, s] + off_v[s]
                return 0

            lax.fori_loop(0, chunk // _L, add_body, 0)
            if c >= 1:
                pb = (c - 1) % 2
                gd[pb].wait()
                gd[pb] = None
                od[pb] = fire_out(c - 1)
            gd[b] = pltpu.async_copy(
                tab_s.at[idx_v.at[b]], ga_v.at[b], gsem)
        lb = (chunks - 1) % 2
        gd[lb].wait()
        od[lb] = fire_out(chunks - 1)
        for b in range(2):
            if od[b] is not None:
                od[b].wait()

    return tok


def kernel(indices, tables):
    B, F = indices.shape
    F2, V, D = tables.shape
    assert F2 == F
    tok = _build(B, F, V, D)
    fq = 128 // D
    Fp = -(-F // fq) * fq
    idxp = jnp.concatenate(
        [indices.astype(jnp.int32),
         jnp.zeros((B, Fp - F), jnp.int32)], axis=1)
    idx1 = (idxp.reshape(B // 8, 8, Fp // fq, fq)
            .transpose(0, 2, 1, 3).reshape(-1))
    j = jnp.arange(4 * 8 * Fp, dtype=jnp.int32)
    f = (j // (8 * fq) % (Fp // fq)) * fq + j % fq
    off = jnp.where(f < F, f * V, 0)
    out2 = tok(idx1, off, tables.reshape(F * V, D))
    out = (out2.reshape(B // 8, Fp // fq, 8, 128)
           .transpose(0, 2, 1, 3).reshape(B, Fp * D)[:, :F * D])
    return out


# speedup vs baseline: 2.1298x; 1.5772x over previous
import functools

import jax
import jax.numpy as jnp
from jax import lax
from jax.experimental import pallas as pl
from jax.experimental.pallas import tpu as pltpu
from jax.experimental.pallas import tpu_sc as plsc

_NC, _NS, _L = 2, 16, 16
_NW = _NC * _NS

_IDX_W = 128


@functools.lru_cache(maxsize=None)
def _build(B, F, V, D):
    rpc = 64
    idxc = rpc * F
    nir = idxc // _IDX_W
    b_per_w = B // _NW
    chunks = b_per_w // rpc
    w_rows = chunks * nir
    assert idxc % _IDX_W == 0 and b_per_w % rpc == 0 and w_rows % 8 == 0

    mesh = plsc.VectorSubcoreMesh(core_axis_name="c", subcore_axis_name="s")

    @functools.partial(
        pl.kernel,
        mesh=mesh,
        compiler_params=pltpu.CompilerParams(use_tc_tiling_on_sc=False),
        out_type=jax.ShapeDtypeStruct((B * F, D), jnp.float32),
        scratch_types=[
            pltpu.VMEM((w_rows, _IDX_W), jnp.int32),
            pltpu.VMEM((nir, _IDX_W), jnp.int32),
            pltpu.VMEM((2, idxc, D), jnp.float32),
            pltpu.VMEM_SHARED((F * V, D), jnp.float32),
            pltpu.SemaphoreType.DMA,
            pltpu.SemaphoreType.DMA,
        ],
    )
    def tok(idx_hbm, off_hbm, tab_hbm, out_hbm, idx_v, off_v, rows_v, tab_s,
            gsem, osem):
        wid = lax.axis_index("s") * _NC + lax.axis_index("c")
        @pl.when(lax.axis_index("s") == 0)
        def _():
            pltpu.sync_copy(tab_hbm, tab_s)
        pltpu.sync_copy(off_hbm, off_v)
        pltpu.sync_copy(idx_hbm.at[pl.ds(wid * w_rows, w_rows)], idx_v)
        plsc.subcore_barrier()
        base_flat = wid * (chunks * idxc)

        def fire_out(c):
            return pltpu.async_copy(
                rows_v.at[c % 2],
                out_hbm.at[pl.ds(base_flat + c * idxc, idxc)],
                osem)

        gd = [None, None]
        od = [None, None]
        for c in range(chunks):
            b = c % 2
            if od[b] is not None:
                od[b].wait()
                od[b] = None
            for j in range(nir):
                r = c * nir + j
                for k in range(_IDX_W // _L):
                    s = pl.ds(k * _L, _L)
                    idx_v[r, s] = idx_v[r, s] + off_v[j, s]
            if c >= 1:
                pb = (c - 1) % 2
                for cp in gd[pb]:
                    cp.wait()
                gd[pb] = None
                od[pb] = fire_out(c - 1)
            gd[b] = [
                pltpu.async_copy(
                    tab_s.at[idx_v.at[c * nir + j]],
                    rows_v.at[b, pl.ds(j * _IDX_W, _IDX_W)],
                    gsem,
                )
                for j in range(nir)
            ]
        lb = (chunks - 1) % 2
        for cp in gd[lb]:
            cp.wait()
        od[lb] = fire_out(chunks - 1)
        for b in range(2):
            if od[b] is not None:
                od[b].wait()

    return tok


def kernel(indices, tables):
    B, F = indices.shape
    F2, V, D = tables.shape
    assert F2 == F
    tok = _build(B, F, V, D)
    nir = (64 * F) // _IDX_W
    off = (((jnp.arange(nir * _IDX_W, dtype=jnp.int32) % F) * V)
           .reshape(nir, _IDX_W))
    idx2 = indices.astype(jnp.int32).reshape((B * F) // _IDX_W, _IDX_W)
    out = tok(idx2, off, tables.reshape(F * V, D))
    return out.reshape(B, F * D)
